# baseline (speedup 1.0000x reference)
import jax
import jax.numpy as jnp
from jax.experimental import pallas as pl
from jax.experimental.pallas import tpu as pltpu

E = 320000
C = 128
CH = 2000
NCHUNK = E // CH
NBUF = 16
NOUTER = NCHUNK // NBUF


def _in_copy(hbm_ref, buf_ref, sem_ref, i, s):
    return pltpu.make_async_copy(
        hbm_ref.at[pl.ds(i * CH, CH), :], buf_ref.at[s], sem_ref.at[s])


def _out_copy(out_hbm, outbuf, sem_ref, i, s):
    return pltpu.make_async_copy(
        outbuf.at[s], out_hbm.at[pl.ds(i, 1), :], sem_ref.at[s])


def _body(linv_hbm, adj_hbm, w_ref, b_ref, out_hbm,
          linv_buf, adj_buf, outbuf, sem_l, sem_a, sem_o):
    w = w_ref[...]
    bias = b_ref[0, 0]

    for s in range(NBUF):
        _in_copy(linv_hbm, linv_buf, sem_l, s, s).start()
        _in_copy(adj_hbm, adj_buf, sem_a, s, s).start()

    def outer(g, carry):
        for s in range(NBUF):
            i = g * NBUF + s
            _in_copy(linv_hbm, linv_buf, sem_l, i, s).wait()
            _in_copy(adj_hbm, adj_buf, sem_a, i, s).wait()

            h = jnp.maximum(adj_buf[s] * linv_buf[s], 0.0)
            res = jax.lax.dot_general(
                w, h, (((1,), (1,)), ((), ())),
                preferred_element_type=jnp.float32,
            ) + bias

            @pl.when(g > 0)
            def _wait_out():
                _out_copy(out_hbm, outbuf, sem_o, i - NBUF, s).wait()

            outbuf[s] = res

            @pl.when(i + NBUF < NCHUNK)
            def _next_in():
                _in_copy(linv_hbm, linv_buf, sem_l, i + NBUF, s).start()
                _in_copy(adj_hbm, adj_buf, sem_a, i + NBUF, s).start()

            _out_copy(out_hbm, outbuf, sem_o, i, s).start()
        return carry

    jax.lax.fori_loop(0, NOUTER, outer, 0)

    for s in range(NBUF):
        _out_copy(out_hbm, outbuf, sem_o, NCHUNK - NBUF + s, s).wait()


def kernel(x_e, Linv, adjacency, W, b):
    del x_e
    out = pl.pallas_call(
        _body,
        in_specs=[
            pl.BlockSpec(memory_space=pltpu.MemorySpace.HBM),
            pl.BlockSpec(memory_space=pltpu.MemorySpace.HBM),
            pl.BlockSpec(memory_space=pltpu.MemorySpace.VMEM),
            pl.BlockSpec(memory_space=pltpu.MemorySpace.VMEM),
        ],
        out_specs=pl.BlockSpec(memory_space=pltpu.MemorySpace.HBM),
        out_shape=jax.ShapeDtypeStruct((NCHUNK, CH), jnp.float32),
        scratch_shapes=[
            pltpu.VMEM((NBUF, CH, C), jnp.float32),
            pltpu.VMEM((NBUF, CH, C), jnp.float32),
            pltpu.VMEM((NBUF, 1, CH), jnp.float32),
            pltpu.SemaphoreType.DMA((NBUF,)),
            pltpu.SemaphoreType.DMA((NBUF,)),
            pltpu.SemaphoreType.DMA((NBUF,)),
        ],
    )(Linv, adjacency, W, b.reshape(1, 1))
    return out.reshape(E, 1)


# speedup vs baseline: 1.0229x; 1.0229x over previous
import jax
import jax.numpy as jnp
from jax.experimental import pallas as pl
from jax.experimental.pallas import tpu as pltpu

E = 320000
C = 128
CH = 2500
NCHUNK = E // CH
NBUF = 8
NOUTER = NCHUNK // NBUF


def _in_copy(hbm_ref, buf_ref, sem_ref, i, s):
    return pltpu.make_async_copy(
        hbm_ref.at[pl.ds(i * CH, CH), :], buf_ref.at[s], sem_ref.at[s])


def _out_copy(out_hbm, outbuf, sem_ref, i, s):
    return pltpu.make_async_copy(
        outbuf.at[s], out_hbm.at[pl.ds(i, 1), :], sem_ref.at[s])


def _body(linv_hbm, adj_hbm, w_ref, b_ref, out_hbm,
          linv_buf, adj_buf, outbuf, sem_l, sem_a, sem_o):
    w = w_ref[...]
    bias = b_ref[0, 0]

    for s in range(NBUF):
        _in_copy(linv_hbm, linv_buf, sem_l, s, s).start()
        _in_copy(adj_hbm, adj_buf, sem_a, s, s).start()

    def outer(g, carry):
        for s in range(NBUF):
            i = g * NBUF + s
            _in_copy(linv_hbm, linv_buf, sem_l, i, s).wait()
            _in_copy(adj_hbm, adj_buf, sem_a, i, s).wait()

            h = jnp.maximum(adj_buf[s] * linv_buf[s], 0.0)
            res = jax.lax.dot_general(
                w, h, (((1,), (1,)), ((), ())),
                preferred_element_type=jnp.float32,
            ) + bias

            @pl.when(g > 0)
            def _wait_out():
                _out_copy(out_hbm, outbuf, sem_o, i - NBUF, s).wait()

            outbuf[s] = res

            @pl.when(i + NBUF < NCHUNK)
            def _next_in():
                _in_copy(linv_hbm, linv_buf, sem_l, i + NBUF, s).start()
                _in_copy(adj_hbm, adj_buf, sem_a, i + NBUF, s).start()

            _out_copy(out_hbm, outbuf, sem_o, i, s).start()
        return carry

    jax.lax.fori_loop(0, NOUTER, outer, 0)

    for s in range(NBUF):
        _out_copy(out_hbm, outbuf, sem_o, NCHUNK - NBUF + s, s).wait()


def kernel(x_e, Linv, adjacency, W, b):
    del x_e
    out = pl.pallas_call(
        _body,
        in_specs=[
            pl.BlockSpec(memory_space=pltpu.MemorySpace.HBM),
            pl.BlockSpec(memory_space=pltpu.MemorySpace.HBM),
            pl.BlockSpec(memory_space=pltpu.MemorySpace.VMEM),
            pl.BlockSpec(memory_space=pltpu.MemorySpace.VMEM),
        ],
        out_specs=pl.BlockSpec(memory_space=pltpu.MemorySpace.HBM),
        out_shape=jax.ShapeDtypeStruct((NCHUNK, CH), jnp.float32),
        scratch_shapes=[
            pltpu.VMEM((NBUF, CH, C), jnp.float32),
            pltpu.VMEM((NBUF, CH, C), jnp.float32),
            pltpu.VMEM((NBUF, 1, CH), jnp.float32),
            pltpu.SemaphoreType.DMA((NBUF,)),
            pltpu.SemaphoreType.DMA((NBUF,)),
            pltpu.SemaphoreType.DMA((NBUF,)),
        ],
    )(Linv, adjacency, W, b.reshape(1, 1))
    return out.reshape(E, 1)
